# Initial kernel scaffold; baseline (speedup 1.0000x reference)
import jax
import jax.numpy as jnp
from jax.experimental import pallas as pl


def kernel(x, table, pos_enc):
    raise NotImplementedError("write your pallas kernel here")


# speedup vs baseline: 2.5572x; 2.5572x over previous
import functools

import jax
import jax.numpy as jnp
from jax import lax
from jax.experimental import pallas as pl
from jax.experimental.pallas import tpu as pltpu
from jax.experimental.pallas import tpu_sc as plsc

_info = plsc.get_sparse_core_info()
_NC, _NS, _L = _info.num_cores, _info.num_subcores, _info.num_lanes
_NW = _NC * _NS


def _build(N, D, SEQ):
    n_per_w = N // _NW
    C = 128
    n_chunks = n_per_w // C
    assert n_per_w % C == 0 and n_chunks % 2 == 0 and D % _L == 0
    coef = 1.0 / (D ** 0.5)
    n_sl = D // _L

    mesh = plsc.VectorSubcoreMesh(core_axis_name="c", subcore_axis_name="s")

    @functools.partial(
        pl.kernel,
        mesh=mesh,
        out_type=jax.ShapeDtypeStruct((N, D), jnp.float32),
        compiler_params=pltpu.CompilerParams(use_tc_tiling_on_sc=False),
        scratch_types=[
            pltpu.VMEM((n_chunks, C), jnp.int32),
            pltpu.VMEM((SEQ, D), jnp.float32),
            pltpu.VMEM((C, D), jnp.float32),
            pltpu.VMEM((C, D), jnp.float32),
            pltpu.SemaphoreType.DMA,
            pltpu.SemaphoreType.DMA,
        ],
    )
    def emb_kernel(x_hbm, table_hbm, pos_hbm, out_hbm,
                   idx_v, pos_v, rows0, rows1, sem0, sem1):
        wid = lax.axis_index("s") * _NC + lax.axis_index("c")
        base = wid * n_per_w
        pltpu.sync_copy(x_hbm.at[wid], idx_v)
        pltpu.sync_copy(pos_hbm, pos_v)
        bufs = (rows0, rows1)
        sems = (sem0, sem1)

        pltpu.async_copy(table_hbm.at[idx_v.at[0]], rows0, sem0)

        def pair_body(p, carry):
            for b in range(2):
                kk = p * 2 + b
                buf, sem = bufs[b], sems[b]
                pltpu.make_async_copy(table_hbm.at[idx_v.at[kk]], buf, sem).wait()
                @pl.when(kk + 1 < n_chunks)
                def _():
                    pltpu.async_copy(
                        table_hbm.at[idx_v.at[kk + 1]], bufs[1 - b], sems[1 - b])
                chunk_base = base + kk * C

                def row_body(r, rcarry):
                    pos_row = lax.rem(chunk_base + r, SEQ)
                    for j in range(n_sl):
                        sl = pl.ds(j * _L, _L)
                        buf[r, sl] = buf[r, sl] * coef + pos_v[pos_row, sl]
                    return rcarry

                lax.fori_loop(0, C, row_body, 0)
                pltpu.sync_copy(buf, out_hbm.at[pl.ds(chunk_base, C)])
            return carry

        lax.fori_loop(0, n_chunks // 2, pair_body, 0)

    return emb_kernel


@jax.jit
def kernel(x, table, pos_enc):
    B, S = x.shape
    V, D = table.shape
    N = B * S
    emb_kernel = _build(N, D, S)
    n_per_w = N // _NW
    C = 128
    xr = x.astype(jnp.int32).reshape(_NW, n_per_w // C, C)
    out = emb_kernel(xr, table, pos_enc)
    return out.reshape(B, S, D)


# speedup vs baseline: 6.3330x; 2.4766x over previous
import functools

import jax
import jax.numpy as jnp
from jax import lax
from jax.experimental import pallas as pl
from jax.experimental.pallas import tpu as pltpu
from jax.experimental.pallas import tpu_sc as plsc

_info = plsc.get_sparse_core_info()
_NC, _NS, _L = _info.num_cores, _info.num_subcores, _info.num_lanes
_NW = _NC * _NS


def _build(N, D, SEQ):
    n_per_w = N // _NW
    C = 128
    n_chunks = n_per_w // C
    assert n_per_w % C == 0 and n_chunks % 2 == 0 and D % _L == 0
    coef = 1.0 / (D ** 0.5)
    n_sl = D // _L

    mesh = plsc.VectorSubcoreMesh(core_axis_name="c", subcore_axis_name="s")

    @functools.partial(
        pl.kernel,
        mesh=mesh,
        out_type=jax.ShapeDtypeStruct((N, D), jnp.float32),
        compiler_params=pltpu.CompilerParams(use_tc_tiling_on_sc=False),
        scratch_types=[
            pltpu.VMEM((n_chunks, C), jnp.int32),
            pltpu.VMEM((SEQ, D), jnp.float32),
            pltpu.VMEM((C, D), jnp.float32),
            pltpu.VMEM((C, D), jnp.float32),
            pltpu.SemaphoreType.DMA,
            pltpu.SemaphoreType.DMA,
            pltpu.SemaphoreType.DMA,
            pltpu.SemaphoreType.DMA,
        ],
    )
    def emb_kernel(x_hbm, table_hbm, pos_hbm, out_hbm,
                   idx_v, pos_v, rows0, rows1, gsem0, gsem1, ssem0, ssem1):
        wid = lax.axis_index("s") * _NC + lax.axis_index("c")
        base = wid * n_per_w
        pltpu.sync_copy(x_hbm.at[wid], idx_v)
        pltpu.sync_copy(pos_hbm, pos_v)
        bufs = (rows0, rows1)
        gsems = (gsem0, gsem1)
        ssems = (ssem0, ssem1)

        def out_slice(kk):
            return out_hbm.at[pl.ds(base + kk * C, C)]

        pltpu.async_copy(table_hbm.at[idx_v.at[0]], rows0, gsem0)

        def pair_body(p, carry):
            for b in range(2):
                kk = p * 2 + b
                buf = bufs[b]
                pltpu.make_async_copy(table_hbm.at[idx_v.at[kk]], buf, gsems[b]).wait()
                @pl.when(kk + 1 < n_chunks)
                def _():
                    @pl.when(kk >= 1)
                    def _():
                        pltpu.make_async_copy(
                            bufs[1 - b], out_slice(kk - 1), ssems[1 - b]).wait()
                    pltpu.async_copy(
                        table_hbm.at[idx_v.at[kk + 1]], bufs[1 - b], gsems[1 - b])
                chunk_base = base + kk * C

                @plsc.parallel_loop(0, C, unroll=4)
                def _(r):
                    pos_row = lax.rem(chunk_base + r, SEQ)
                    for j in range(n_sl):
                        sl = pl.ds(j * _L, _L)
                        buf[r, sl] = buf[r, sl] * coef + pos_v[pos_row, sl]

                pltpu.async_copy(buf, out_slice(kk), ssems[b])
            return carry

        lax.fori_loop(0, n_chunks // 2, pair_body, 0)
        pltpu.make_async_copy(bufs[0], out_slice(n_chunks - 2), ssems[0]).wait()
        pltpu.make_async_copy(bufs[1], out_slice(n_chunks - 1), ssems[1]).wait()

    return emb_kernel


@jax.jit
def kernel(x, table, pos_enc):
    B, S = x.shape
    V, D = table.shape
    N = B * S
    emb_kernel = _build(N, D, S)
    n_per_w = N // _NW
    C = 128
    xr = x.astype(jnp.int32).reshape(_NW, n_per_w // C, C)
    out = emb_kernel(xr, table, pos_enc)
    return out.reshape(B, S, D)
